# Initial kernel scaffold; baseline (speedup 1.0000x reference)
import jax
import jax.numpy as jnp
from jax.experimental import pallas as pl


def kernel(x, index):
    raise NotImplementedError("write your pallas kernel here")


# speedup vs baseline: 1.6874x; 1.6874x over previous
import functools

import jax
import jax.numpy as jnp
from jax import lax
from jax.experimental import pallas as pl
from jax.experimental.pallas import tpu as pltpu
from jax.experimental.pallas import tpu_sc as plsc

N = 320000
D = 128
S = 10000
NC = 2
NS = 16
NW = NC * NS
RW = N // NW
C = 128
NCH = RW // C
TAIL = RW - NCH * C
C1 = 80
NCH1 = RW // C1
SPLITS = (0, 21, 42, 63, 84, 105, 125)
NP = len(SPLITS) - 1
SP = 10240
ZR = SP // NS
SEG_W = SP // NW

_mesh = plsc.VectorSubcoreMesh(core_axis_name="c", subcore_axis_name="s")


def _worker_id():
    return lax.axis_index("c") * NS + lax.axis_index("s")


def _make_k1(lo, hi):
    def body(x_hbm, idx_hbm, sums_hbm, xbuf, idx_v, acc_sh):
        cid = lax.axis_index("c")
        sid = lax.axis_index("s")
        w = _worker_id()
        zbase = sid * ZR
        obase = cid * SP + sid * ZR

        def zrow(r, carry):
            for d in range(D // 16):
                xbuf[r, pl.ds(16 * d, 16)] = jnp.zeros((16,), jnp.float32)
            return carry

        lax.fori_loop(0, C1, zrow, 0)
        for j in range(ZR // C1):
            pltpu.sync_copy(xbuf, acc_sh.at[pl.ds(zbase + j * C1, C1)])
        plsc.subcore_barrier()

        row0 = w * RW
        for k in range(lo, hi):
            base = row0 + k * C1
            pltpu.sync_copy(idx_hbm.at[pl.ds(base, C1)], idx_v)
            pltpu.sync_copy(x_hbm.at[pl.ds(base, C1)], xbuf)
            pltpu.sync_copy(xbuf, acc_sh.at[idx_v], add=True)
        plsc.subcore_barrier()

        pltpu.sync_copy(acc_sh.at[pl.ds(sid * ZR, ZR)],
                        sums_hbm.at[pl.ds(obase, ZR)])

    return pl.kernel(
        body,
        out_type=jax.ShapeDtypeStruct((NC * SP, D), jnp.float32),
        mesh=_mesh,
        scratch_types=[
            pltpu.VMEM((C1, D), jnp.float32),
            pltpu.VMEM((C1,), jnp.int32),
            pltpu.VMEM_SHARED((SP, D), jnp.float32),
        ],
    )


_K1_STAGES = [_make_k1(SPLITS[j], SPLITS[j + 1]) for j in range(NP)]


def _make_k1c(lo, hi):
    def body(idx_hbm, cnts_hbm, ones_v, idx_v, cnt_sh):
        cid = lax.axis_index("c")
        sid = lax.axis_index("s")
        w = _worker_id()
        zbase = sid * ZR
        obase = cid * SP + sid * ZR

        def zrow(r, carry):
            for d in range(D // 16):
                ones_v[r, pl.ds(16 * d, 16)] = jnp.zeros((16,), jnp.float32)
            return carry

        lax.fori_loop(0, C1, zrow, 0)
        for j in range(ZR // C1):
            pltpu.sync_copy(ones_v, cnt_sh.at[pl.ds(zbase + j * C1, C1)])

        def orow(r, carry):
            for d in range(D // 16):
                ones_v[r, pl.ds(16 * d, 16)] = jnp.ones((16,), jnp.float32)
            return carry

        lax.fori_loop(0, C1, orow, 0)
        plsc.subcore_barrier()

        row0 = w * RW
        for k in range(lo, hi):
            base = row0 + k * C1
            pltpu.sync_copy(idx_hbm.at[pl.ds(base, C1)], idx_v)
            pltpu.sync_copy(ones_v, cnt_sh.at[idx_v], add=True)
        plsc.subcore_barrier()

        pltpu.sync_copy(cnt_sh.at[pl.ds(sid * ZR, ZR)],
                        cnts_hbm.at[pl.ds(obase, ZR)])

    return pl.kernel(
        body,
        out_type=jax.ShapeDtypeStruct((NC * SP, D), jnp.float32),
        mesh=_mesh,
        scratch_types=[
            pltpu.VMEM((C1, D), jnp.float32),
            pltpu.VMEM((C1,), jnp.int32),
            pltpu.VMEM_SHARED((SP, D), jnp.float32),
        ],
    )


_K1C_STAGES = [_make_k1c(SPLITS[j], SPLITS[j + 1]) for j in range(NP)]


@functools.partial(
    pl.kernel,
    out_type=jax.ShapeDtypeStruct((SP, D), jnp.float32),
    mesh=_mesh,
    scratch_types=[
        pltpu.VMEM((64, D), jnp.float32),
        pltpu.VMEM((64, D), jnp.float32),
        pltpu.VMEM((64, D), jnp.float32),
    ],
)
def _k2_means(*args):
    sums = args[:NP]
    cnts = args[NP:2 * NP]
    mean_hbm = args[2 * NP]
    p0, c0, pt = args[2 * NP + 1:]
    w = _worker_id()

    def block(j, carry):
        seg0 = w * SEG_W + j * 64
        pltpu.sync_copy(sums[0].at[pl.ds(seg0, 64)], p0)
        pltpu.sync_copy(cnts[0].at[pl.ds(seg0, 64)], c0)

        def acc_slab(src, dst, off):
            pltpu.sync_copy(src.at[pl.ds(off + seg0, 64)], pt)

            def arow(r, inner):
                for d in range(D // 16):
                    sl = pl.ds(16 * d, 16)
                    dst[r, sl] = dst[r, sl] + pt[r, sl]
                return inner

            lax.fori_loop(0, 64, arow, 0)

        acc_slab(sums[0], p0, SP)
        acc_slab(cnts[0], c0, SP)
        for jj in range(1, NP):
            acc_slab(sums[jj], p0, 0)
            acc_slab(sums[jj], p0, SP)
            acc_slab(cnts[jj], c0, 0)
            acc_slab(cnts[jj], c0, SP)

        def seg(s, inner):
            recip = 1.0 / jnp.maximum(c0[s, pl.ds(0, 16)], 1.0)
            for d in range(D // 16):
                sl = pl.ds(16 * d, 16)
                p0[s, sl] = p0[s, sl] * recip
            return inner

        lax.fori_loop(0, 64, seg, 0)
        pltpu.sync_copy(p0, mean_hbm.at[pl.ds(seg0, 64)])
        return carry

    lax.fori_loop(0, SEG_W // 64, block, 0)


@functools.partial(
    pl.kernel,
    out_type=jax.ShapeDtypeStruct((N, D), jnp.float32),
    mesh=_mesh,
    scratch_types=[
        pltpu.VMEM((C,), jnp.int32),
        pltpu.VMEM((C, D), jnp.float32),
        pltpu.VMEM((C, D), jnp.float32),
        pltpu.VMEM((TAIL,), jnp.int32),
        pltpu.VMEM((TAIL, D), jnp.float32),
        pltpu.VMEM((TAIL, D), jnp.float32),
        pltpu.SemaphoreType.DMA,
    ],
)
def _k3_subtract(x_hbm, idx_hbm, mean_hbm, out_hbm,
                 idx_v, xbuf, mbuf, idxt, xtail, mtail, sem):
    w = _worker_id()
    row0 = w * RW

    def do_chunk(base, idx_ref, x_ref, m_ref, rows):
        pltpu.sync_copy(idx_hbm.at[pl.ds(base, rows)], idx_ref)
        gather = pltpu.async_copy(mean_hbm.at[idx_ref], m_ref, sem)
        pltpu.sync_copy(x_hbm.at[pl.ds(base, rows)], x_ref)
        gather.wait()

        def row(r, carry):
            for d in range(D // 16):
                sl = pl.ds(16 * d, 16)
                x_ref[r, sl] = x_ref[r, sl] - m_ref[r, sl]
            return carry

        lax.fori_loop(0, rows, row, 0)
        pltpu.sync_copy(x_ref, out_hbm.at[pl.ds(base, rows)])

    def chunk(k, carry):
        do_chunk(row0 + k * C, idx_v, xbuf, mbuf, C)
        return carry

    lax.fori_loop(0, NCH, chunk, 0)
    do_chunk(row0 + NCH * C, idxt, xtail, mtail, TAIL)


def kernel(x, index):
    idx32 = index.astype(jnp.int32)
    sums = [stage(x, idx32) for stage in _K1_STAGES]
    cnts = [stage(idx32) for stage in _K1C_STAGES]
    mean = _k2_means(*sums, *cnts)
    return _k3_subtract(x, idx32, mean)
